# Initial kernel scaffold; baseline (speedup 1.0000x reference)
import jax
import jax.numpy as jnp
from jax.experimental import pallas as pl


def kernel(points, params):
    raise NotImplementedError("write your pallas kernel here")


# speedup vs baseline: 8.2644x; 8.2644x over previous
import functools
import numpy as np
import jax
import jax.numpy as jnp
from jax import lax
from jax.experimental import pallas as pl
from jax.experimental.pallas import tpu as pltpu
from jax.experimental.pallas import tpu_sc as plsc

_K = 16


def _chain(x, layers):
    rows, _ = x.shape
    relus = tuple(bool(r) for (_, _, r) in layers)
    cout = layers[-1][0].shape[1]
    tile = min(rows, 512)

    def body(x_ref, *refs):
        out_ref = refs[-1]
        h = x_ref[...]
        for i, rl in enumerate(relus):
            w = refs[2 * i][...]
            b = refs[2 * i + 1][...]
            h = jnp.dot(h, w, preferred_element_type=jnp.float32) + b
            if rl:
                h = jnp.maximum(h, 0.0)
        out_ref[...] = h

    in_specs = [pl.BlockSpec((tile, x.shape[1]), lambda r: (r, 0))]
    args = [x]
    for (w, b, _) in layers:
        in_specs.append(pl.BlockSpec(w.shape, lambda r: (0, 0)))
        in_specs.append(pl.BlockSpec((1, b.shape[0]), lambda r: (0, 0)))
        args.append(w)
        args.append(b.reshape(1, -1))
    return pl.pallas_call(
        body,
        grid=(rows // tile,),
        in_specs=in_specs,
        out_specs=pl.BlockSpec((tile, cout), lambda r: (r, 0)),
        out_shape=jax.ShapeDtypeStruct((rows, cout), jnp.float32),
    )(*args)


def _fps(pos, m):
    bsz, n, _ = pos.shape
    s = n // 128
    posr = pos.transpose(0, 2, 1).reshape(bsz, 3, s, 128)

    def body(p_ref, o_ref):
        x = p_ref[0, 0, :, :]
        y = p_ref[0, 1, :, :]
        z = p_ref[0, 2, :, :]
        ii = (lax.broadcasted_iota(jnp.int32, (s, 128), 0) * 128
              + lax.broadcasted_iota(jnp.int32, (s, 128), 1))

        def step(i, carry):
            d, last = carry
            oh = ii == last
            xl = jnp.sum(jnp.where(oh, x, 0.0))
            yl = jnp.sum(jnp.where(oh, y, 0.0))
            zl = jnp.sum(jnp.where(oh, z, 0.0))
            row = jnp.concatenate(
                [xl.reshape(1, 1), yl.reshape(1, 1), zl.reshape(1, 1)], axis=1)
            o_ref[0, pl.ds(i, 1), :] = row
            dd = (x - xl) ** 2 + (y - yl) ** 2 + (z - zl) ** 2
            d = jnp.minimum(d, dd)
            mx = jnp.max(d)
            cand = jnp.where(d >= mx, ii, jnp.int32(1 << 30))
            nxt = jnp.min(cand)
            return d, nxt

        d0 = jnp.full((s, 128), 1e10, jnp.float32)
        lax.fori_loop(0, m, step, (d0, jnp.int32(0)))

    return pl.pallas_call(
        body,
        grid=(bsz,),
        in_specs=[pl.BlockSpec((1, 3, s, 128), lambda b: (b, 0, 0, 0))],
        out_specs=pl.BlockSpec((1, m, 3), lambda b: (b, 0, 0)),
        out_shape=jax.ShapeDtypeStruct((bsz, m, 3), jnp.float32),
    )(posr)


def _knn(qpos, kposT):
    bsz, mq, _ = qpos.shape
    n = kposT.shape[2]
    r = min(mq, 256)

    def body(q_ref, kt_ref, idx_ref):
        q = q_ref[0]
        kt = kt_ref[0]
        x = q[:, 0:1]; y = q[:, 1:2]; z = q[:, 2:3]
        qn = (x * x + y * y) + z * z
        kx = kt[0:1, :]; ky = kt[1:2, :]; kz = kt[2:3, :]
        kn = (kx * kx + ky * ky) + kz * kz
        qb = q.astype(jnp.bfloat16).astype(jnp.float32)
        kb = kt.astype(jnp.bfloat16).astype(jnp.float32)
        px = qb[:, 0:1] * kb[0:1, :]
        py = qb[:, 1:2] * kb[1:2, :]
        pz = qb[:, 2:3] * kb[2:3, :]
        s = px + py
        bv = s - px
        e1 = (px - (s - bv)) + (py - bv)
        t = s + pz
        bv2 = t - s
        e2 = (s - (t - bv2)) + (pz - bv2)
        qk = t + (e1 + e2)
        d2 = (qn + kn) - 2.0 * qk
        d = jnp.sqrt(jnp.maximum(d2, 1e-12))
        jj = lax.broadcasted_iota(jnp.int32, (r, n), 1)
        base = (pl.program_id(0) * n).astype(jnp.int32)
        big = jnp.float32(3.0e38)
        idxs = []
        for _ in range(_K):
            mn = jnp.min(d, axis=1, keepdims=True)
            cand = jnp.where(d <= mn, jj, jnp.int32(n))
            ci = jnp.min(cand, axis=1, keepdims=True)
            oh = jj == ci
            d = jnp.where(oh, big, d)
            idxs.append(ci)
        idx_ref[0] = jnp.concatenate(idxs, axis=1) + base

    idx = pl.pallas_call(
        body,
        grid=(bsz, mq // r),
        in_specs=[pl.BlockSpec((1, r, 3), lambda b, t: (b, t, 0)),
                  pl.BlockSpec((1, 3, n), lambda b, t: (b, 0, 0))],
        out_specs=pl.BlockSpec((1, r, _K), lambda b, t: (b, t, 0)),
        out_shape=jax.ShapeDtypeStruct((bsz, mq, _K), jnp.int32),
    )(qpos, kposT)
    return idx.transpose(0, 2, 1)


def _sc_gather(table, idx, width):
    num = idx.shape[0]
    window = 128
    idx2 = idx.reshape(1, num)
    mesh = plsc.VectorSubcoreMesh(core_axis_name="c", subcore_axis_name="s")

    def kernel(x_hbm, i_hbm, o_hbm):
        def gbody(i_vmem, o_vmem):
            pltpu.sync_copy(x_hbm.at[i_vmem.at[0]], o_vmem)

        pltpu.emit_pipeline(
            gbody,
            grid=(num // window,),
            in_specs=[pl.BlockSpec((1, window), lambda i: (0, i))],
            out_specs=[pl.BlockSpec((window, width), lambda i: (i, 0))],
            core_axis_name=("c", "s"),
            dimension_semantics=(pltpu.PARALLEL,),
        )(i_hbm, o_hbm)

    f = pl.kernel(
        kernel,
        out_type=jax.ShapeDtypeStruct((num, width), jnp.float32),
        mesh=mesh,
    )
    return f(table, idx2)


def _pad_table(x, width):
    rows, c = x.shape
    if c == width:
        return x
    return jnp.concatenate(
        [x, jnp.zeros((rows, width - c), jnp.float32)], axis=1)


def _pt_combine(g3, kf, pos, p, dim, wtab):
    bsz = g3.shape[0]
    mq = g3.shape[2]
    r = min(mq, 256)
    ph = p["pos1"]["w"].shape[1]
    ah = p["attn1"]["w"].shape[1]

    ws = [p["pos1"]["w"], p["pos1"]["b"].reshape(1, -1),
          p["pos2"]["w"], p["pos2"]["b"].reshape(1, -1),
          p["attn1"]["w"], p["attn1"]["b"].reshape(1, -1),
          p["attn2"]["w"], p["attn2"]["b"].reshape(1, -1),
          p["linear2"]["w"], p["linear2"]["b"].reshape(1, -1)]

    def body(g_ref, k_ref, pos_ref, *refs):
        (p1w, p1b, p2w, p2b, a1w, a1b, a2w, a2b, l2w, l2b, o_ref) = refs
        g = g_ref[0].reshape(_K * r, wtab)
        kf_t = k_ref[0]
        pos_t = pos_ref[0]

        def mm(a, b):
            return jnp.dot(a, b, preferred_element_type=jnp.float32)

        def tile16(x):
            return jnp.concatenate([x] * _K, axis=0)

        q_g = g[:, 0:dim]
        pos_g = g[:, dim:dim + 3]
        v_g = g[:, dim + 3:2 * dim + 3]

        pt1 = tile16(mm(pos_t, p1w[...]))
        h1 = jnp.maximum(mm(pos_g, p1w[...]) - pt1 + p1b[...], 0.0)
        pe = jnp.maximum(mm(h1, p2w[...]) + p2b[...], 0.0)
        ka = tile16(mm(kf_t, a1w[...]))
        a = jnp.maximum(
            mm(q_g, a1w[...]) + mm(pe, a1w[...]) - ka + a1b[...], 0.0)
        e = jnp.maximum(mm(a, a2w[...]) + a2b[...], 0.0)

        e3 = e.reshape(_K, r, dim)
        mx = jnp.max(e3, axis=0)
        ex3 = jnp.exp(e3 - mx[None, :, :])
        den = jnp.sum(ex3, axis=0)
        v3 = v_g.reshape(_K, r, dim)
        num = jnp.sum(v3 * ex3, axis=0)
        out = num / den
        o_ref[0] = mm(out, l2w[...]) + l2b[...]

    in_specs = [
        pl.BlockSpec((1, _K, r, wtab), lambda b, t: (b, 0, t, 0)),
        pl.BlockSpec((1, r, dim), lambda b, t: (b, t, 0)),
        pl.BlockSpec((1, r, 3), lambda b, t: (b, t, 0)),
    ]
    args = [g3, kf, pos]
    for c in ws:
        in_specs.append(pl.BlockSpec(c.shape, lambda b, t: (0, 0)))
        args.append(c)
    return pl.pallas_call(
        body,
        grid=(bsz, mq // r),
        in_specs=in_specs,
        out_specs=pl.BlockSpec((1, r, dim), lambda b, t: (b, t, 0)),
        out_shape=jax.ShapeDtypeStruct((bsz, mq, dim), jnp.float32),
    )(*args)


def _down_combine(g3, c):
    bsz = g3.shape[0]
    mq = g3.shape[2]
    wtab = g3.shape[3]
    r = min(mq, 256)

    def body(g_ref, o_ref):
        g = g_ref[0]
        t = g[0, :, 0:c]
        for l in range(1, _K):
            t = jnp.maximum(t, g[l, :, 0:c])
        o_ref[0] = t

    return pl.pallas_call(
        body,
        grid=(bsz, mq // r),
        in_specs=[pl.BlockSpec((1, _K, r, wtab), lambda b, t: (b, 0, t, 0))],
        out_specs=pl.BlockSpec((1, r, c), lambda b, t: (b, t, 0)),
        out_shape=jax.ShapeDtypeStruct((bsz, mq, c), jnp.float32),
    )(g3)


def _up_combine(g3, posf, w, b, c):
    bsz = g3.shape[0]
    mq = g3.shape[2]
    wtab = g3.shape[3]
    cout = w.shape[1]
    r = min(mq, 256)
    b2 = b.reshape(1, -1)

    def body(g_ref, pos_ref, w_r, b_r, o_ref):
        g = g_ref[0]
        pos_t = pos_ref[0]
        num = jnp.zeros((r, c), jnp.float32)
        den = jnp.zeros((r, 1), jnp.float32)
        for l in range(_K):
            f_l = g[l, :, 0:c]
            p_l = g[l, :, c:c + 3]
            diff = p_l - pos_t
            d2 = jnp.sum(diff * diff, axis=1, keepdims=True)
            d = jnp.sqrt(jnp.maximum(d2, 1e-12))
            wgt = 1.0 / (d + 1e-6)
            num = num + wgt * f_l
            den = den + wgt
        nf = num / den
        y = jnp.dot(nf, w_r[...], preferred_element_type=jnp.float32) + b_r[...]
        o_ref[0] = jnp.maximum(y, 0.0)

    return pl.pallas_call(
        body,
        grid=(bsz, mq // r),
        in_specs=[
            pl.BlockSpec((1, _K, r, wtab), lambda bb, t: (bb, 0, t, 0)),
            pl.BlockSpec((1, r, 3), lambda bb, t: (bb, t, 0)),
            pl.BlockSpec(w.shape, lambda bb, t: (0, 0)),
            pl.BlockSpec((1, cout), lambda bb, t: (0, 0)),
        ],
        out_specs=pl.BlockSpec((1, r, cout), lambda bb, t: (bb, t, 0)),
        out_shape=jax.ShapeDtypeStruct((bsz, mq, cout), jnp.float32),
    )(g3, posf, w, b2)


def _posT(pos):
    return pos.transpose(0, 2, 1)


def _pt_block(p, feature, pos, dim):
    bsz, npt, cin = feature.shape
    wtab = 128 if 2 * dim + 3 <= 128 else 256
    wqkv = jnp.concatenate([p["q"]["w"], p["k"]["w"], p["v"]["w"]], axis=1)
    bqkv = jnp.concatenate([p["q"]["b"], p["k"]["b"], p["v"]["b"]], axis=0)
    qkv = _chain(feature.reshape(bsz * npt, cin),
                 [(p["linear1"]["w"], p["linear1"]["b"], True),
                  (wqkv, bqkv, True)])
    q = qkv[:, 0:dim]
    kf = qkv[:, dim:2 * dim].reshape(bsz, npt, dim)
    v = qkv[:, 2 * dim:3 * dim]
    pos2 = pos.reshape(bsz * npt, 3)
    table = _pad_table(jnp.concatenate([q, pos2, v], axis=1), wtab)
    idx = _knn(pos, _posT(pos))
    g = _sc_gather(table, idx.reshape(-1), wtab)
    g3 = g.reshape(bsz, _K, npt, wtab)
    return _pt_combine(g3, kf, pos, p, dim, wtab)


def _forward(points, params):
    bsz, n, _ = points.shape
    pos = points[:, :, 0:3]
    feat = points[:, :, 3:9]

    out1 = _chain(feat.reshape(bsz * n, 6),
                  [(params["mlp1"]["l1"]["w"], params["mlp1"]["l1"]["b"], True),
                   (params["mlp1"]["l2"]["w"], params["mlp1"]["l2"]["b"], False)]
                  ).reshape(bsz, n, 32)

    def down(pd, featd, w, b, cout):
        bb, nn, cc = featd.shape
        m = nn // 4
        g = _chain(featd.reshape(bb * nn, cc), [(w, b, True)])
        spos = _fps(pd, m)
        idx = _knn(spos, _posT(pd))
        gg = _sc_gather(_pad_table(g, 128), idx.reshape(-1), 128)
        out = _down_combine(gg.reshape(bb, _K, m, 128), cout)
        return out, spos

    def up(featc, posc, posf, w, b):
        bb, nc, cc = featc.shape
        idx = _knn(posf, _posT(posc))
        table = _pad_table(
            jnp.concatenate([featc.reshape(bb * nc, cc),
                             posc.reshape(bb * nc, 3)], axis=1), 128)
        gg = _sc_gather(table, idx.reshape(-1), 128)
        return _up_combine(gg.reshape(bb, _K, posf.shape[1], 128),
                           posf, w, b, cc)

    od1, pd1 = down(pos, out1, params["ds1"]["w"], params["ds1"]["b"], 16)
    opd1 = _pt_block(params["ptd1"], od1, pd1, 16)
    od2, pd2 = down(pd1, opd1, params["ds2"]["w"], params["ds2"]["b"], 32)
    opd2 = _pt_block(params["ptd2"], od2, pd2, 32)
    od3, pd3 = down(pd2, opd2, params["ds3"]["w"], params["ds3"]["b"], 64)
    opd3 = _pt_block(params["ptd3"], od3, pd3, 64)

    ou1 = up(opd3, pd3, pd2, params["us1"]["w"], params["us1"]["b"])
    opu1 = _pt_block(params["ptu1"], ou1, pd2, 32)
    ou2 = up(opu1, pd2, pd1, params["us2"]["w"], params["us2"]["b"])
    opu2 = _pt_block(params["ptu2"], ou2, pd1, 16)
    ou3 = up(opu2, pd1, pos, params["us3"]["w"], params["us3"]["b"])
    opu3 = _pt_block(params["ptu3"], ou3, pos, 8)

    out2 = _chain(
        opu3.reshape(bsz * n, 8),
        [(params["mlp2"]["l1"]["w"], params["mlp2"]["l1"]["b"], True),
         (params["mlp2"]["l2"]["w"], params["mlp2"]["l2"]["b"], False),
         (params["mlp3"]["l1"]["w"], params["mlp3"]["l1"]["b"], True),
         (params["mlp3"]["l2"]["w"], params["mlp3"]["l2"]["b"], False)])
    return out2.reshape(bsz, n, 13)


def kernel(points, params):
    return _forward(points, params)
